# Initial kernel scaffold; baseline (speedup 1.0000x reference)
import functools

import jax
import jax.numpy as jnp
from jax import lax
from jax.experimental import pallas as pl
from jax.experimental.pallas import tpu as pltpu
from jax.experimental.pallas import tpu_sc as plsc

N_DRUG = 5000
N_DIS = 5000
H = 128
E = 320000
E_LABEL = 320000

NC = 2
NT = 16
NW = NC * NT
TILE_E = E // NW
CHUNK = 128
NFULL = TILE_E // CHUNK
TAIL = TILE_E - NFULL * CHUNK
CW = 16
RSTRIPE = 312

_MESH = plsc.VectorSubcoreMesh(core_axis_name="c", subcore_axis_name="s")
_F32 = jnp.float32


def _zero_fill(ref):
    z = jnp.zeros((16,), _F32)

    @pl.loop(0, ref.shape[0])
    def _(r):
        @pl.loop(0, ref.shape[1], step=16)
        def _(c):
            ref[r, pl.ds(c, 16)] = z


def _seg_body(do_counts, edges, xf, xr, *refs):
    if do_counts:
        (aggf_hbm, aggr_hbm, cntf_hbm, cntr_hbm,
         idx_v, idxt_v, rows_v, zb_v, zbc_v, ones_v,
         aggf_s, aggr_s, cntf_s, cntr_s, sem) = refs
    else:
        (aggf_hbm, aggr_hbm,
         idx_v, idxt_v, rows_v, zb_v,
         aggf_s, aggr_s, sem) = refs

    cid = lax.axis_index("c")
    sid = lax.axis_index("s")
    wid = cid * NT + sid

    _zero_fill(zb_v)
    if do_counts:
        _zero_fill(zbc_v)

        @pl.loop(0, ones_v.shape[0])
        def _(r):
            ones_v[r, pl.ds(0, CW)] = jnp.ones((CW,), _F32)

    rb = sid * RSTRIPE

    @pl.loop(0, RSTRIPE, step=8)
    def _(r):
        pltpu.sync_copy(zb_v, aggf_s.at[pl.ds(rb + r, 8)])
        pltpu.sync_copy(zb_v, aggr_s.at[pl.ds(rb + r, 8)])
        if do_counts:
            pltpu.sync_copy(zbc_v, cntf_s.at[pl.ds(rb + r, 8)])
            pltpu.sync_copy(zbc_v, cntr_s.at[pl.ds(rb + r, 8)])

    @pl.when(sid == NT - 1)
    def _():
        last = NT * RSTRIPE
        pltpu.sync_copy(zb_v, aggf_s.at[pl.ds(last, 8)])
        pltpu.sync_copy(zb_v, aggr_s.at[pl.ds(last, 8)])
        if do_counts:
            pltpu.sync_copy(zbc_v, cntf_s.at[pl.ds(last, 8)])
            pltpu.sync_copy(zbc_v, cntr_s.at[pl.ds(last, 8)])

    plsc.subcore_barrier()

    ebase = wid * TILE_E

    @pl.loop(0, NFULL)
    def _(ch):
        base = ebase + ch * CHUNK
        pltpu.sync_copy(edges.at[:, pl.ds(base, CHUNK)], idx_v)
        src = idx_v.at[0]
        dst = idx_v.at[1]
        pltpu.async_copy(xf.at[src], rows_v, sem).wait()
        pltpu.sync_copy(rows_v, aggf_s.at[dst], add=True)
        pltpu.async_copy(xr.at[dst], rows_v, sem).wait()
        pltpu.sync_copy(rows_v, aggr_s.at[src], add=True)
        if do_counts:
            pltpu.sync_copy(ones_v, cntf_s.at[dst], add=True)
            pltpu.sync_copy(ones_v, cntr_s.at[src], add=True)

    tbase = ebase + NFULL * CHUNK
    pltpu.sync_copy(edges.at[:, pl.ds(tbase, TAIL)], idxt_v)
    srct = idxt_v.at[0]
    dstt = idxt_v.at[1]
    rows_t = rows_v.at[pl.ds(0, TAIL)]
    pltpu.async_copy(xf.at[srct], rows_t, sem).wait()
    pltpu.sync_copy(rows_t, aggf_s.at[dstt], add=True)
    pltpu.async_copy(xr.at[dstt], rows_t, sem).wait()
    pltpu.sync_copy(rows_t, aggr_s.at[srct], add=True)
    if do_counts:
        ones_t = ones_v.at[pl.ds(0, TAIL)]
        pltpu.sync_copy(ones_t, cntf_s.at[dstt], add=True)
        pltpu.sync_copy(ones_t, cntr_s.at[srct], add=True)

    plsc.subcore_barrier()

    pltpu.sync_copy(aggf_s.at[pl.ds(rb, RSTRIPE)],
                    aggf_hbm.at[cid, pl.ds(rb, RSTRIPE)])
    pltpu.sync_copy(aggr_s.at[pl.ds(rb, RSTRIPE)],
                    aggr_hbm.at[cid, pl.ds(rb, RSTRIPE)])
    if do_counts:
        pltpu.sync_copy(cntf_s.at[pl.ds(rb, RSTRIPE)],
                        cntf_hbm.at[cid, pl.ds(rb, RSTRIPE)])
        pltpu.sync_copy(cntr_s.at[pl.ds(rb, RSTRIPE)],
                        cntr_hbm.at[cid, pl.ds(rb, RSTRIPE)])

    @pl.when(sid == NT - 1)
    def _():
        last = NT * RSTRIPE
        pltpu.sync_copy(aggf_s.at[pl.ds(last, 8)],
                        aggf_hbm.at[cid, pl.ds(last, 8)])
        pltpu.sync_copy(aggr_s.at[pl.ds(last, 8)],
                        aggr_hbm.at[cid, pl.ds(last, 8)])
        if do_counts:
            pltpu.sync_copy(cntf_s.at[pl.ds(last, 8)],
                            cntf_hbm.at[cid, pl.ds(last, 8)])
            pltpu.sync_copy(cntr_s.at[pl.ds(last, 8)],
                            cntr_hbm.at[cid, pl.ds(last, 8)])


def _make_seg(do_counts):
    outs = [jax.ShapeDtypeStruct((NC, N_DIS, H), _F32),
            jax.ShapeDtypeStruct((NC, N_DRUG, H), _F32)]
    scratch = [
        pltpu.VMEM((2, CHUNK), jnp.int32),
        pltpu.VMEM((2, TAIL), jnp.int32),
        pltpu.VMEM((CHUNK, H), _F32),
        pltpu.VMEM((8, H), _F32),
    ]
    if do_counts:
        outs += [jax.ShapeDtypeStruct((NC, N_DIS, CW), _F32),
                 jax.ShapeDtypeStruct((NC, N_DRUG, CW), _F32)]
        scratch += [pltpu.VMEM((8, CW), _F32),
                    pltpu.VMEM((CHUNK, CW), _F32)]
    shared = [pltpu.VMEM_SHARED((N_DIS, H), _F32),
              pltpu.VMEM_SHARED((N_DRUG, H), _F32)]
    if do_counts:
        shared += [pltpu.VMEM_SHARED((N_DIS, CW), _F32),
                   pltpu.VMEM_SHARED((N_DRUG, CW), _F32)]
    return pl.kernel(
        functools.partial(_seg_body, do_counts),
        out_type=tuple(outs),
        mesh=_MESH,
        scratch_types=scratch + shared + [pltpu.SemaphoreType.DMA],
    )


_seg_counts = _make_seg(True)
_seg_plain = _make_seg(False)


def _cls_body(labels, od, oz, p_hbm, idx_v, idxt_v, g0_v, g1_v, p_v, sem):
    cid = lax.axis_index("c")
    sid = lax.axis_index("s")
    wid = cid * NT + sid
    ebase = wid * TILE_E

    def dot_rows(n, g0, g1, out):
        @pl.loop(0, n)
        def _(e):
            acc = g0[e, pl.ds(0, 16)] * g1[e, pl.ds(0, 16)]
            for j in range(1, H // 16):
                acc = acc + g0[e, pl.ds(j * 16, 16)] * g1[e, pl.ds(j * 16, 16)]
            out[e, pl.ds(0, CW)] = acc

    @pl.loop(0, NFULL)
    def _(ch):
        base = ebase + ch * CHUNK
        pltpu.sync_copy(labels.at[:, pl.ds(base, CHUNK)], idx_v)
        pltpu.async_copy(od.at[idx_v.at[0]], g0_v, sem).wait()
        pltpu.async_copy(oz.at[idx_v.at[1]], g1_v, sem).wait()
        dot_rows(CHUNK, g0_v, g1_v, p_v)
        pltpu.sync_copy(p_v, p_hbm.at[pl.ds(base, CHUNK)])

    tbase = ebase + NFULL * CHUNK
    pltpu.sync_copy(labels.at[:, pl.ds(tbase, TAIL)], idxt_v)
    g0t = g0_v.at[pl.ds(0, TAIL)]
    g1t = g1_v.at[pl.ds(0, TAIL)]
    pltpu.async_copy(od.at[idxt_v.at[0]], g0t, sem).wait()
    pltpu.async_copy(oz.at[idxt_v.at[1]], g1t, sem).wait()
    dot_rows(TAIL, g0_v, g1_v, p_v)
    pltpu.sync_copy(p_v.at[pl.ds(0, TAIL)], p_hbm.at[pl.ds(tbase, TAIL)])


_classifier = pl.kernel(
    _cls_body,
    out_type=jax.ShapeDtypeStruct((E_LABEL, CW), _F32),
    mesh=_MESH,
    scratch_types=[
        pltpu.VMEM((2, CHUNK), jnp.int32),
        pltpu.VMEM((2, TAIL), jnp.int32),
        pltpu.VMEM((CHUNK, H), _F32),
        pltpu.VMEM((CHUNK, H), _F32),
        pltpu.VMEM((CHUNK, CW), _F32),
        pltpu.SemaphoreType.DMA,
    ],
)


def _xdis_body(dx_ref, lw_ref, lb_ref, me_ref, out_ref):
    out_ref[...] = (dx_ref[...] * lw_ref[...] + lb_ref[...][None, :]
                    + me_ref[...])


def _tc_xdis(disease_x, lin_W, lin_b, movie_emb):
    return pl.pallas_call(
        _xdis_body,
        out_shape=jax.ShapeDtypeStruct((N_DIS, H), _F32),
    )(disease_x, lin_W, lin_b, movie_emb)


def _mean(agg_ref, cnt_ref):
    s = agg_ref[0] + agg_ref[1]
    c = cnt_ref[0, :, 0:1] + cnt_ref[1, :, 0:1]
    return s / jnp.maximum(c, 1.0)


def _layer_body(relu, aggf_ref, cntf_ref, aggr_ref, cntr_ref,
                xdis_ref, xdrug_ref, wlf_ref, blf_ref, wrf_ref,
                wlr_ref, blr_ref, wrr_ref, hdis_ref, hdrug_ref):
    mf = _mean(aggf_ref, cntf_ref)
    mr = _mean(aggr_ref, cntr_ref)
    hp = lax.Precision.HIGHEST
    hdis = (jnp.dot(mf, wlf_ref[...], precision=hp) + blf_ref[...][None, :]
            + jnp.dot(xdis_ref[...], wrf_ref[...], precision=hp))
    hdrug = (jnp.dot(mr, wlr_ref[...], precision=hp) + blr_ref[...][None, :]
             + jnp.dot(xdrug_ref[...], wrr_ref[...], precision=hp))
    if relu:
        hdis = jnp.maximum(hdis, 0.0)
        hdrug = jnp.maximum(hdrug, 0.0)
    hdis_ref[...] = hdis
    hdrug_ref[...] = hdrug


def _tc_layer(relu, aggf, cntf, aggr, cntr, xdis, xdrug,
              wlf, blf, wrf, wlr, blr, wrr):
    return pl.pallas_call(
        functools.partial(_layer_body, relu),
        out_shape=(jax.ShapeDtypeStruct((N_DIS, H), _F32),
                   jax.ShapeDtypeStruct((N_DRUG, H), _F32)),
    )(aggf, cntf, aggr, cntr, xdis, xdrug, wlf, blf, wrf, wlr, blr, wrr)


def _reduce_body(p_ref, out_ref):
    out_ref[...] = jnp.sum(p_ref[...], axis=1)


def _tc_reduce(p):
    return pl.pallas_call(
        _reduce_body,
        out_shape=jax.ShapeDtypeStruct((E_LABEL,), _F32),
    )(p)


def kernel(drug_node_id, disease_node_id, disease_x, edge_index,
           edge_label_index, user_emb, movie_emb, lin_W, lin_b,
           Wl1f, bl1f, Wr1f, Wl1r, bl1r, Wr1r,
           Wl2f, bl2f, Wr2f, Wl2r, bl2r, Wr2r):
    x_drug = user_emb
    x_dis = _tc_xdis(disease_x, lin_W, lin_b, movie_emb)

    aggf1, aggr1, cntf, cntr = _seg_counts(edge_index, x_drug, x_dis)
    h_dis, h_drug = _tc_layer(True, aggf1, cntf, aggr1, cntr, x_dis, x_drug,
                              Wl1f, bl1f, Wr1f, Wl1r, bl1r, Wr1r)

    aggf2, aggr2 = _seg_plain(edge_index, h_drug, h_dis)
    o_dis, o_drug = _tc_layer(False, aggf2, cntf, aggr2, cntr, h_dis, h_drug,
                              Wl2f, bl2f, Wr2f, Wl2r, bl2r, Wr2r)

    p = _classifier(edge_label_index, o_drug, o_dis)
    return _tc_reduce(p)


# speedup vs baseline: 2.7488x; 2.7488x over previous
import functools

import jax
import jax.numpy as jnp
from jax import lax
from jax.experimental import pallas as pl
from jax.experimental.pallas import tpu as pltpu
from jax.experimental.pallas import tpu_sc as plsc

N_DRUG = 5000
N_DIS = 5000
H = 128
E = 320000
E_LABEL = 320000

NC = 2
NT = 16
NW = NC * NT
CHUNK = 128
NCHUNK = E // CHUNK
CW = 16
RSTRIPE = 312

_MESH = plsc.VectorSubcoreMesh(core_axis_name="c", subcore_axis_name="s")
_F32 = jnp.float32


NPAD = 5120
NBLK = NPAD // CHUNK


def _zero_fill(ref):
    z = jnp.zeros((16,), _F32)

    @pl.loop(0, ref.shape[0])
    def _(r):
        @pl.loop(0, ref.shape[1], step=16)
        def _(c):
            ref[r, pl.ds(c, 16)] = z


def _fill_iota(zidx_v, base):
    lanes = lax.iota(jnp.int32, 16)
    for k in range(CHUNK // 16):
        zidx_v[0, pl.ds(k * 16, 16)] = lanes + (base + k * 16)


def _seg_body(do_counts, edges, xf, xr, *refs):
    if do_counts:
        (aggf_hbm, aggr_hbm, cntf_hbm, cntr_hbm,
         idx_v, zidx_v, rows_v, ones_v,
         aggf_s, aggr_s, cntf_s, cntr_s, sem) = refs
    else:
        (aggf_hbm, aggr_hbm,
         idx_v, zidx_v, rows_v,
         aggf_s, aggr_s, sem) = refs

    cid = lax.axis_index("c")
    sid = lax.axis_index("s")
    wid = cid * NT + sid

    _zero_fill(rows_v)
    if do_counts:
        _zero_fill(ones_v)
    nblk = NBLK // NT + jnp.where(sid < NBLK % NT, 1, 0)

    @pl.loop(0, nblk)
    def _(u):
        b = (u * NT + sid) * CHUNK
        _fill_iota(zidx_v, b)
        pltpu.sync_copy(rows_v, aggf_s.at[zidx_v.at[0]])
        pltpu.sync_copy(rows_v, aggr_s.at[zidx_v.at[0]])
        if do_counts:
            pltpu.sync_copy(ones_v, cntf_s.at[zidx_v.at[0]])
            pltpu.sync_copy(ones_v, cntr_s.at[zidx_v.at[0]])

    if do_counts:
        one = jnp.ones((16,), _F32)

        @pl.loop(0, CHUNK)
        def _(r):
            ones_v[r, pl.ds(0, CW)] = one

    plsc.subcore_barrier()

    nch = NCHUNK // NW + jnp.where(wid < NCHUNK % NW, 1, 0)

    @pl.loop(0, nch)
    def _(ch):
        base = (ch * NW + wid) * CHUNK
        pltpu.sync_copy(edges.at[:, pl.ds(base, CHUNK)], idx_v)
        src = idx_v.at[0]
        dst = idx_v.at[1]
        pltpu.async_copy(xf.at[src], rows_v, sem).wait()
        pltpu.sync_copy(rows_v, aggf_s.at[dst], add=True)
        pltpu.async_copy(xr.at[dst], rows_v, sem).wait()
        pltpu.sync_copy(rows_v, aggr_s.at[src], add=True)
        if do_counts:
            pltpu.sync_copy(ones_v, cntf_s.at[dst], add=True)
            pltpu.sync_copy(ones_v, cntr_s.at[src], add=True)

    plsc.subcore_barrier()

    @pl.loop(0, nblk)
    def _(u):
        b = (u * NT + sid) * CHUNK
        _fill_iota(zidx_v, b)
        pltpu.sync_copy(aggf_s.at[zidx_v.at[0]], rows_v)
        pltpu.sync_copy(rows_v, aggf_hbm.at[cid, pl.ds(b, CHUNK)])
        pltpu.sync_copy(aggr_s.at[zidx_v.at[0]], rows_v)
        pltpu.sync_copy(rows_v, aggr_hbm.at[cid, pl.ds(b, CHUNK)])
        if do_counts:
            pltpu.sync_copy(cntf_s.at[zidx_v.at[0]], ones_v)
            pltpu.sync_copy(ones_v, cntf_hbm.at[cid, pl.ds(b, CHUNK)])
            pltpu.sync_copy(cntr_s.at[zidx_v.at[0]], ones_v)
            pltpu.sync_copy(ones_v, cntr_hbm.at[cid, pl.ds(b, CHUNK)])


def _make_seg(do_counts):
    outs = [jax.ShapeDtypeStruct((NC, NPAD, H), _F32),
            jax.ShapeDtypeStruct((NC, NPAD, H), _F32)]
    scratch = [
        pltpu.VMEM((2, CHUNK), jnp.int32),
        pltpu.VMEM((1, CHUNK), jnp.int32),
        pltpu.VMEM((CHUNK, H), _F32),
    ]
    if do_counts:
        outs += [jax.ShapeDtypeStruct((NC, NPAD, CW), _F32),
                 jax.ShapeDtypeStruct((NC, NPAD, CW), _F32)]
        scratch += [pltpu.VMEM((CHUNK, CW), _F32)]
    shared = [pltpu.VMEM_SHARED((NPAD, H), _F32),
              pltpu.VMEM_SHARED((NPAD, H), _F32)]
    if do_counts:
        shared += [pltpu.VMEM_SHARED((NPAD, CW), _F32),
                   pltpu.VMEM_SHARED((NPAD, CW), _F32)]
    return pl.kernel(
        functools.partial(_seg_body, do_counts),
        out_type=tuple(outs),
        mesh=_MESH,
        scratch_types=scratch + shared + [pltpu.SemaphoreType.DMA],
    )


_seg_counts = _make_seg(True)
_seg_plain = _make_seg(False)


def _cls_body(labels, od, oz, p_hbm, idx_v, g0_v, g1_v, p_v, sem):
    cid = lax.axis_index("c")
    sid = lax.axis_index("s")
    wid = cid * NT + sid
    nch = NCHUNK // NW + jnp.where(wid < NCHUNK % NW, 1, 0)

    @pl.loop(0, nch)
    def _(ch):
        base = (ch * NW + wid) * CHUNK
        pltpu.sync_copy(labels.at[:, pl.ds(base, CHUNK)], idx_v)
        pltpu.async_copy(od.at[idx_v.at[0]], g0_v, sem).wait()
        pltpu.async_copy(oz.at[idx_v.at[1]], g1_v, sem).wait()

        @pl.loop(0, CHUNK)
        def _(e):
            acc = g0_v[e, pl.ds(0, 16)] * g1_v[e, pl.ds(0, 16)]
            for j in range(1, H // 16):
                acc = acc + (g0_v[e, pl.ds(j * 16, 16)]
                             * g1_v[e, pl.ds(j * 16, 16)])
            p_v[e, pl.ds(0, CW)] = acc

        pltpu.sync_copy(p_v, p_hbm.at[pl.ds(base, CHUNK)])


_classifier = pl.kernel(
    _cls_body,
    out_type=jax.ShapeDtypeStruct((E_LABEL, CW), _F32),
    mesh=_MESH,
    scratch_types=[
        pltpu.VMEM((2, CHUNK), jnp.int32),
        pltpu.VMEM((CHUNK, H), _F32),
        pltpu.VMEM((CHUNK, H), _F32),
        pltpu.VMEM((CHUNK, CW), _F32),
        pltpu.SemaphoreType.DMA,
    ],
)


def _xdis_body(dx_ref, lw_ref, lb_ref, me_ref, out_ref):
    out_ref[...] = (dx_ref[...] * lw_ref[...] + lb_ref[...][None, :]
                    + me_ref[...])


def _tc_xdis(disease_x, lin_W, lin_b, movie_emb):
    return pl.pallas_call(
        _xdis_body,
        out_shape=jax.ShapeDtypeStruct((N_DIS, H), _F32),
    )(disease_x, lin_W, lin_b, movie_emb)


def _mean(agg_ref, cnt_ref, n):
    s = agg_ref[0, :n] + agg_ref[1, :n]
    c = cnt_ref[0, :n, 0:1] + cnt_ref[1, :n, 0:1]
    return s / jnp.maximum(c, 1.0)


def _layer_body(relu, aggf_ref, cntf_ref, aggr_ref, cntr_ref,
                xdis_ref, xdrug_ref, wlf_ref, blf_ref, wrf_ref,
                wlr_ref, blr_ref, wrr_ref, hdis_ref, hdrug_ref):
    mf = _mean(aggf_ref, cntf_ref, N_DIS)
    mr = _mean(aggr_ref, cntr_ref, N_DRUG)
    hp = lax.Precision.HIGHEST
    hdis = (jnp.dot(mf, wlf_ref[...], precision=hp) + blf_ref[...][None, :]
            + jnp.dot(xdis_ref[...], wrf_ref[...], precision=hp))
    hdrug = (jnp.dot(mr, wlr_ref[...], precision=hp) + blr_ref[...][None, :]
             + jnp.dot(xdrug_ref[...], wrr_ref[...], precision=hp))
    if relu:
        hdis = jnp.maximum(hdis, 0.0)
        hdrug = jnp.maximum(hdrug, 0.0)
    hdis_ref[...] = hdis
    hdrug_ref[...] = hdrug


def _tc_layer(relu, aggf, cntf, aggr, cntr, xdis, xdrug,
              wlf, blf, wrf, wlr, blr, wrr):
    return pl.pallas_call(
        functools.partial(_layer_body, relu),
        out_shape=(jax.ShapeDtypeStruct((N_DIS, H), _F32),
                   jax.ShapeDtypeStruct((N_DRUG, H), _F32)),
    )(aggf, cntf, aggr, cntr, xdis, xdrug, wlf, blf, wrf, wlr, blr, wrr)


def _reduce_body(p_ref, out_ref):
    lanes = lax.broadcasted_iota(jnp.int32, (H, H), 0)
    cols = lax.broadcasted_iota(jnp.int32, (H, H), 1)
    sel = jnp.where((lanes // CW == cols) & (cols < 8), 1.0, 0.0)
    out_ref[...] = jnp.dot(p_ref[...], sel.astype(_F32),
                           precision=lax.Precision.HIGHEST)


def _tc_reduce(p):
    rows = E_LABEL // 8
    blk = 2000
    p2 = p.reshape(rows, H)
    out = pl.pallas_call(
        _reduce_body,
        grid=(rows // blk,),
        in_specs=[pl.BlockSpec((blk, H), lambda i: (i, 0))],
        out_specs=pl.BlockSpec((blk, H), lambda i: (i, 0)),
        out_shape=jax.ShapeDtypeStruct((rows, H), _F32),
    )(p2)
    return out[:, :8].reshape(E_LABEL)


ABLK = 200


def _agg_body(a_ref, xf_ref, xr_ref, aggf_ref, aggr_ref, cf_ref, cr_ref):
    i = pl.program_id(0)
    hp = lax.Precision.HIGHEST
    a_blk = a_ref[...]
    aggf_ref[...] = jnp.dot(a_blk, xf_ref[...], precision=hp)
    cf_ref[...] = jnp.sum(a_blk, axis=1, keepdims=True)

    @pl.when(i == 0)
    def _():
        aggr_ref[...] = jnp.zeros_like(aggr_ref)
        cr_ref[...] = jnp.zeros_like(cr_ref)

    aggr_ref[...] += lax.dot_general(a_blk, xr_ref[...],
                                     (((0,), (0,)), ((), ())), precision=hp)
    cr_ref[...] += jnp.sum(a_blk, axis=0, keepdims=True)


def _agg_pass(a, xf, xr):
    n = a.shape[0]
    grid = (n // ABLK,)
    return pl.pallas_call(
        _agg_body,
        grid=grid,
        in_specs=[
            pl.BlockSpec((ABLK, n), lambda i: (i, 0)),
            pl.BlockSpec((n, H), lambda i: (0, 0)),
            pl.BlockSpec((ABLK, H), lambda i: (i, 0)),
        ],
        out_specs=(
            pl.BlockSpec((ABLK, H), lambda i: (i, 0)),
            pl.BlockSpec((n, H), lambda i: (0, 0)),
            pl.BlockSpec((ABLK, 1), lambda i: (i, 0)),
            pl.BlockSpec((1, n), lambda i: (0, 0)),
        ),
        out_shape=(
            jax.ShapeDtypeStruct((n, H), _F32),
            jax.ShapeDtypeStruct((n, H), _F32),
            jax.ShapeDtypeStruct((n, 1), _F32),
            jax.ShapeDtypeStruct((1, n), _F32),
        ),
    )(a, xf, xr)


def _layer2_body(relu, aggf_ref, cf_ref, aggr_ref, cr_ref,
                 xdis_ref, xdrug_ref, wlf_ref, blf_ref, wrf_ref,
                 wlr_ref, blr_ref, wrr_ref, hdis_ref, hdrug_ref):
    hp = lax.Precision.HIGHEST
    mf = aggf_ref[...] / jnp.maximum(cf_ref[...], 1.0)
    mr = aggr_ref[...] / jnp.maximum(cr_ref[...], 1.0)
    hdis = (jnp.dot(mf, wlf_ref[...], precision=hp) + blf_ref[...][None, :]
            + jnp.dot(xdis_ref[...], wrf_ref[...], precision=hp))
    hdrug = (jnp.dot(mr, wlr_ref[...], precision=hp) + blr_ref[...][None, :]
             + jnp.dot(xdrug_ref[...], wrr_ref[...], precision=hp))
    if relu:
        hdis = jnp.maximum(hdis, 0.0)
        hdrug = jnp.maximum(hdrug, 0.0)
    hdis_ref[...] = hdis
    hdrug_ref[...] = hdrug


def _tc_layer2(relu, aggf, cf, aggr, cr, xdis, xdrug,
               wlf, blf, wrf, wlr, blr, wrr):
    return pl.pallas_call(
        functools.partial(_layer2_body, relu),
        out_shape=(jax.ShapeDtypeStruct((N_DIS, H), _F32),
                   jax.ShapeDtypeStruct((N_DRUG, H), _F32)),
    )(aggf, cf, aggr, cr, xdis, xdrug, wlf, blf, wrf, wlr, blr, wrr)


def kernel(drug_node_id, disease_node_id, disease_x, edge_index,
           edge_label_index, user_emb, movie_emb, lin_W, lin_b,
           Wl1f, bl1f, Wr1f, Wl1r, bl1r, Wr1r,
           Wl2f, bl2f, Wr2f, Wl2r, bl2r, Wr2r):
    x_drug = user_emb
    x_dis = _tc_xdis(disease_x, lin_W, lin_b, movie_emb)

    a = jnp.zeros((N_DIS, N_DRUG), _F32).at[edge_index[1], edge_index[0]].add(1.0)

    aggf1, aggr1, cf, cr = _agg_pass(a, x_drug, x_dis)
    cr = cr.reshape(N_DRUG, 1)
    h_dis, h_drug = _tc_layer2(True, aggf1, cf, aggr1, cr, x_dis, x_drug,
                               Wl1f, bl1f, Wr1f, Wl1r, bl1r, Wr1r)

    aggf2, aggr2, _, _ = _agg_pass(a, h_drug, h_dis)
    o_dis, o_drug = _tc_layer2(False, aggf2, cf, aggr2, cr, h_dis, h_drug,
                               Wl2f, bl2f, Wr2f, Wl2r, bl2r, Wr2r)

    p = _classifier(edge_label_index, o_drug, o_dis)
    return _tc_reduce(p)


def _xla_seg_plain(edge_index, xf, xr):
    src, dst = edge_index[0], edge_index[1]
    aggf = jax.ops.segment_sum(xf[src], dst, num_segments=NPAD)
    aggr = jax.ops.segment_sum(xr[dst], src, num_segments=NPAD)
    z = jnp.zeros_like(aggf)
    return (jnp.stack([aggf, z]), jnp.stack([aggr, z]))


def _xla_seg_counts(edge_index, xf, xr):
    aggf, aggr = _xla_seg_plain(edge_index, xf, xr)
    src, dst = edge_index[0], edge_index[1]
    ones = jnp.ones((E,), _F32)
    cf = jax.ops.segment_sum(ones, dst, num_segments=NPAD)
    cr = jax.ops.segment_sum(ones, src, num_segments=NPAD)
    cf = jnp.broadcast_to(cf[:, None], (NPAD, CW))
    cr = jnp.broadcast_to(cr[:, None], (NPAD, CW))
    zc = jnp.zeros_like(cf)
    return aggf, aggr, jnp.stack([cf, zc]), jnp.stack([cr, zc])


def _xla_classifier(edge_label_index, od, oz):
    g0 = od[edge_label_index[0]]
    g1 = oz[edge_label_index[1]]
    return (g0 * g1).reshape(E_LABEL, 8, CW).sum(axis=1)
